# baseline (speedup 1.0000x reference)
import functools

import jax
import jax.numpy as jnp
from jax import lax
from jax.experimental import pallas as pl
from jax.experimental.pallas import tpu as pltpu
from jax.experimental.pallas import tpu_sc as plsc

N_ROWS = 320000
D = 128
N_SEG = 1024
NC = 2
NS = 16
NW = NC * NS
GROUP = 128
N_GROUPS = N_ROWS // GROUP
GROUPS_PER_W = N_GROUPS // NW
N_EXTRA = N_GROUPS - GROUPS_PER_W * NW
IDX_PAD_ROWS = 2504
IDX_WIN = 88
SEG_PER_TILE = N_SEG // NS
HROWS = N_SEG // 8
HSEG_PER_TILE = HROWS // NS
N_PAIRS = GROUPS_PER_W // 2


def _sc_body(x_hbm, b_hbm, sums_hbm, cnts_hbm,
             acc, cntsq, xbuf0, xbuf1, xbuf2, xbuf3, idxall, hist, idbuf,
             fzero, gsem0, gsem1, gsem2, gsem3,
             ssem0, ssem1, ssem2, ssem3):
    c = lax.axis_index("c")
    s = lax.axis_index("s")
    wid = c * NS + s

    zeros16 = jnp.zeros((16,), jnp.float32)
    zeros16i = jnp.zeros((16,), jnp.int32)
    iota16 = lax.iota(jnp.int32, 16)

    start = GROUPS_PER_W * wid + jnp.minimum(wid, N_EXTRA)
    has_extra = wid < N_EXTRA
    off = start & 7
    wstart = pl.multiple_of(start - off, 8)

    bufs = (xbuf0, xbuf1, xbuf2, xbuf3)
    gsems = (gsem0, gsem1, gsem2, gsem3)
    ssems = (ssem0, ssem1, ssem2, ssem3)

    def _gather(k, b):
        pltpu.async_copy(
            x_hbm.at[pl.ds((start + k) * GROUP, GROUP)], bufs[b], gsems[b])

    def _drain_gather(b):
        pltpu.make_async_copy(x_hbm.at[pl.ds(0, GROUP)], bufs[b],
                              gsems[b]).wait()

    def _scatter(k, b):
        pltpu.async_copy(bufs[b], acc.at[idxall.at[off + k]], ssems[b],
                         add=True)

    def _drain_scatter(b):
        pltpu.make_async_copy(x_hbm.at[pl.ds(0, GROUP)], bufs[b],
                              ssems[b]).wait()

    for b in range(4):
        _gather(b, b)

    def _z_hist(k, _):
        hist[k // 8, pl.ds((k % 8) * 16, 16)] = zeros16i
        return 0
    lax.fori_loop(0, HROWS * 8, _z_hist, 0)

    def _z_fzero(k, _):
        fzero[k // 8, pl.ds((k % 8) * 16, 16)] = zeros16
        return 0
    lax.fori_loop(0, SEG_PER_TILE * 8, _z_fzero, 0)

    for j in range(8):
        idbuf[0, pl.ds(j * 16, 16)] = iota16 + (j * 16)

    seg0 = s * SEG_PER_TILE
    hseg0 = s * HSEG_PER_TILE
    pltpu.sync_copy(fzero, acc.at[pl.ds(seg0, SEG_PER_TILE)])
    pltpu.sync_copy(hist.at[pl.ds(0, HSEG_PER_TILE)],
                    cntsq.at[pl.ds(hseg0, HSEG_PER_TILE)])
    plsc.subcore_barrier()

    pltpu.sync_copy(b_hbm.at[pl.ds(wstart, IDX_WIN)], idxall)

    ones16 = zeros16i + 1

    def _hist_row(r):
        for j in range(8):
            v = idxall[off + r, pl.ds(j * 16, 16)]
            plsc.addupdate_scatter(hist, [v >> 3, (v & 7) * 16], ones16)

    def _quad(g, _):
        k0 = 4 * g
        for b in range(4):
            _hist_row(k0 + b)
            _drain_gather(b)
            _scatter(k0 + b, b)
        for b in range(4):
            _drain_scatter(b)
            _gather(k0 + b + 4, b)
        return 0
    lax.fori_loop(0, (GROUPS_PER_W - 6) // 4, _quad, 0)

    t0 = ((GROUPS_PER_W - 6) // 4) * 4
    for b in range(4):
        _hist_row(t0 + b)
        _drain_gather(b)
        _scatter(t0 + b, b)
    for b in range(2):
        _drain_scatter(b)
        _gather(t0 + 4 + b, b)
    for b in (2, 3):
        _drain_scatter(b)
    for b in range(2):
        _hist_row(t0 + 4 + b)
        _drain_gather(b)
        _scatter(t0 + 4 + b, b)
    _drain_scatter(0)

    @pl.when(has_extra)
    def _extra_gather():
        _gather(GROUPS_PER_W, 0)

    _drain_scatter(1)

    @pl.when(has_extra)
    def _extra_scatter():
        _hist_row(GROUPS_PER_W)
        _drain_gather(0)
        _scatter(GROUPS_PER_W, 0)
        _drain_scatter(0)

    pltpu.sync_copy(hist, cntsq.at[idbuf.at[0]], add=True)

    plsc.subcore_barrier()

    pltpu.sync_copy(acc.at[pl.ds(seg0, SEG_PER_TILE)],
                    sums_hbm.at[c, pl.ds(seg0, SEG_PER_TILE)])
    pltpu.sync_copy(cntsq.at[pl.ds(hseg0, HSEG_PER_TILE)],
                    cnts_hbm.at[c, pl.ds(hseg0, HSEG_PER_TILE)])


_sc_segment_sum = functools.partial(
    pl.kernel,
    out_type=(
        jax.ShapeDtypeStruct((NC, N_SEG, D), jnp.float32),
        jax.ShapeDtypeStruct((NC, HROWS, 128), jnp.int32),
    ),
    mesh=plsc.VectorSubcoreMesh(core_axis_name="c", subcore_axis_name="s"),
    scratch_types=[
        pltpu.VMEM_SHARED((N_SEG, D), jnp.float32),
        pltpu.VMEM_SHARED((HROWS, 128), jnp.int32),
        pltpu.VMEM((GROUP, D), jnp.float32),
        pltpu.VMEM((GROUP, D), jnp.float32),
        pltpu.VMEM((GROUP, D), jnp.float32),
        pltpu.VMEM((GROUP, D), jnp.float32),
        pltpu.VMEM((IDX_WIN, 128), jnp.int32),
        pltpu.VMEM((HROWS, 128), jnp.int32),
        pltpu.VMEM((1, 128), jnp.int32),
        pltpu.VMEM((SEG_PER_TILE, 128), jnp.float32),
        pltpu.SemaphoreType.DMA,
        pltpu.SemaphoreType.DMA,
        pltpu.SemaphoreType.DMA,
        pltpu.SemaphoreType.DMA,
        pltpu.SemaphoreType.DMA,
        pltpu.SemaphoreType.DMA,
        pltpu.SemaphoreType.DMA,
        pltpu.SemaphoreType.DMA,
    ],
    compiler_params=pltpu.CompilerParams(needs_layout_passes=False),
)(_sc_body)


def _combine_body(s_ref, c_ref, o_ref):
    sm = s_ref[...]
    cn = c_ref[...]
    tot = sm[0] + sm[1]
    cnt = jnp.maximum((cn[0] + cn[1]).astype(jnp.float32), 1.0)
    o_ref[...] = tot / cnt[:, None]


def _combine(sums, counts):
    return pl.pallas_call(
        _combine_body,
        out_shape=jax.ShapeDtypeStruct((N_SEG, D), jnp.float32),
    )(sums, counts)


def kernel(x, batch):
    b32 = batch.astype(jnp.int32)
    pad = jnp.zeros((IDX_PAD_ROWS * GROUP - N_ROWS,), jnp.int32)
    batch2d = jnp.concatenate([b32, pad]).reshape(IDX_PAD_ROWS, GROUP)
    sums, cnts = _sc_segment_sum(x, batch2d)
    counts = cnts[:, :, ::16].reshape(NC, N_SEG)
    return _combine(sums, counts)


# speedup vs baseline: 1.0102x; 1.0102x over previous
import functools

import jax
import jax.numpy as jnp
from jax import lax
from jax.experimental import pallas as pl
from jax.experimental.pallas import tpu as pltpu
from jax.experimental.pallas import tpu_sc as plsc

N_ROWS = 320000
D = 128
N_SEG = 1024
NC = 2
NS = 16
NW = NC * NS
GROUP = 128
N_GROUPS = N_ROWS // GROUP
SC_GROUPS_PER_W = 64
SC_GROUPS = SC_GROUPS_PER_W * NW
SC_ROWS = SC_GROUPS * GROUP
TC_ROWS = N_ROWS - SC_ROWS
TC_BLK = 512
TC_NBLK = TC_ROWS // TC_BLK
HGROUPS_PER_W = N_GROUPS // NW
N_EXTRA = N_GROUPS - HGROUPS_PER_W * NW
IDX_PAD_ROWS = 2504
IDX_WIN = 88
SEG_PER_TILE = N_SEG // NS
HROWS = N_SEG // 8
HSEG_PER_TILE = HROWS // NS
N_QUAD = SC_GROUPS_PER_W // 4 - 1


def _sc_body(x_hbm, b_hbm, sums_hbm, cnts_hbm,
             acc, cntsq, xbuf0, xbuf1, xbuf2, xbuf3, idxs, idxh, hist,
             idbuf, fzero, gsem0, gsem1, gsem2, gsem3,
             ssem0, ssem1, ssem2, ssem3):
    c = lax.axis_index("c")
    s = lax.axis_index("s")
    wid = c * NS + s

    zeros16 = jnp.zeros((16,), jnp.float32)
    zeros16i = jnp.zeros((16,), jnp.int32)
    iota16 = lax.iota(jnp.int32, 16)

    sstart = pl.multiple_of(SC_GROUPS_PER_W * wid, 8)
    hstart = HGROUPS_PER_W * wid + jnp.minimum(wid, N_EXTRA)
    has_extra = wid < N_EXTRA
    nh = HGROUPS_PER_W + has_extra.astype(jnp.int32)
    hoff = hstart & 7
    hwstart = pl.multiple_of(hstart - hoff, 8)

    bufs = (xbuf0, xbuf1, xbuf2, xbuf3)
    gsems = (gsem0, gsem1, gsem2, gsem3)
    ssems = (ssem0, ssem1, ssem2, ssem3)

    def _gather(k, b):
        pltpu.async_copy(
            x_hbm.at[pl.ds((sstart + k) * GROUP, GROUP)], bufs[b],
            gsems[b])

    def _drain_gather(b):
        pltpu.make_async_copy(x_hbm.at[pl.ds(0, GROUP)], bufs[b],
                              gsems[b]).wait()

    def _scatter(k, b):
        pltpu.async_copy(bufs[b], acc.at[idxs.at[k]], ssems[b], add=True)

    def _drain_scatter(b):
        pltpu.make_async_copy(x_hbm.at[pl.ds(0, GROUP)], bufs[b],
                              ssems[b]).wait()

    for b in range(4):
        _gather(b, b)

    def _z_hist(k, _):
        hist[k // 8, pl.ds((k % 8) * 16, 16)] = zeros16i
        return 0
    lax.fori_loop(0, HROWS * 8, _z_hist, 0)

    def _z_fzero(k, _):
        fzero[k // 8, pl.ds((k % 8) * 16, 16)] = zeros16
        return 0
    lax.fori_loop(0, SEG_PER_TILE * 8, _z_fzero, 0)

    for j in range(8):
        idbuf[0, pl.ds(j * 16, 16)] = iota16 + (j * 16)

    seg0 = s * SEG_PER_TILE
    hseg0 = s * HSEG_PER_TILE
    pltpu.sync_copy(fzero, acc.at[pl.ds(seg0, SEG_PER_TILE)])
    pltpu.sync_copy(hist.at[pl.ds(0, HSEG_PER_TILE)],
                    cntsq.at[pl.ds(hseg0, HSEG_PER_TILE)])
    plsc.subcore_barrier()

    pltpu.sync_copy(b_hbm.at[pl.ds(sstart, SC_GROUPS_PER_W)], idxs)
    pltpu.sync_copy(b_hbm.at[pl.ds(hwstart, IDX_WIN)], idxh)

    ones16 = zeros16i + 1

    def _hist_row(r):
        for j in range(8):
            v = idxh[hoff + r, pl.ds(j * 16, 16)]
            plsc.addupdate_scatter(hist, [v >> 3, (v & 7) * 16], ones16)

    def _quad(g, _):
        k0 = 4 * g
        for b in range(4):
            _hist_row(k0 + b)
            _drain_gather(b)
            _scatter(k0 + b, b)
        for b in range(4):
            _drain_scatter(b)
            _gather(k0 + b + 4, b)
        return 0
    lax.fori_loop(0, N_QUAD, _quad, 0)

    t0 = N_QUAD * 4
    for b in range(4):
        _hist_row(t0 + b)
        _drain_gather(b)
        _scatter(t0 + b, b)
    for b in range(4):
        _drain_scatter(b)

    def _hist_rest(r, _):
        _hist_row(r)
        return 0
    lax.fori_loop(SC_GROUPS_PER_W, nh, _hist_rest, 0)

    pltpu.sync_copy(hist, cntsq.at[idbuf.at[0]], add=True)

    plsc.subcore_barrier()

    pltpu.sync_copy(acc.at[pl.ds(seg0, SEG_PER_TILE)],
                    sums_hbm.at[c, pl.ds(seg0, SEG_PER_TILE)])
    pltpu.sync_copy(cntsq.at[pl.ds(hseg0, HSEG_PER_TILE)],
                    cnts_hbm.at[c, pl.ds(hseg0, HSEG_PER_TILE)])


_sc_segment_sum = functools.partial(
    pl.kernel,
    out_type=(
        jax.ShapeDtypeStruct((NC, N_SEG, D), jnp.float32),
        jax.ShapeDtypeStruct((NC, HROWS, 128), jnp.int32),
    ),
    mesh=plsc.VectorSubcoreMesh(core_axis_name="c", subcore_axis_name="s"),
    scratch_types=[
        pltpu.VMEM_SHARED((N_SEG, D), jnp.float32),
        pltpu.VMEM_SHARED((HROWS, 128), jnp.int32),
        pltpu.VMEM((GROUP, D), jnp.float32),
        pltpu.VMEM((GROUP, D), jnp.float32),
        pltpu.VMEM((GROUP, D), jnp.float32),
        pltpu.VMEM((GROUP, D), jnp.float32),
        pltpu.VMEM((SC_GROUPS_PER_W, 128), jnp.int32),
        pltpu.VMEM((IDX_WIN, 128), jnp.int32),
        pltpu.VMEM((HROWS, 128), jnp.int32),
        pltpu.VMEM((1, 128), jnp.int32),
        pltpu.VMEM((SEG_PER_TILE, 128), jnp.float32),
        pltpu.SemaphoreType.DMA,
        pltpu.SemaphoreType.DMA,
        pltpu.SemaphoreType.DMA,
        pltpu.SemaphoreType.DMA,
        pltpu.SemaphoreType.DMA,
        pltpu.SemaphoreType.DMA,
        pltpu.SemaphoreType.DMA,
        pltpu.SemaphoreType.DMA,
    ],
    compiler_params=pltpu.CompilerParams(needs_layout_passes=False),
)(_sc_body)


def _tc_body(ids_ref, x_ref, o_ref):
    i = pl.program_id(0)

    @pl.when(i == 0)
    def _init():
        o_ref[...] = jnp.zeros_like(o_ref)

    ids = ids_ref[0, 0]
    oh = (jax.lax.broadcasted_iota(jnp.int32, (TC_BLK, N_SEG), 1)
          == ids[:, None]).astype(jnp.bfloat16)
    xb = x_ref[...].astype(jnp.bfloat16)
    o_ref[...] += lax.dot_general(
        oh, xb, (((0,), (0,)), ((), ())),
        preferred_element_type=jnp.float32)


def _tc_partial(ids3d, x_tc):
    return pl.pallas_call(
        _tc_body,
        grid=(TC_NBLK,),
        in_specs=[
            pl.BlockSpec((1, 1, TC_BLK), lambda i: (i, 0, 0)),
            pl.BlockSpec((TC_BLK, D), lambda i: (i, 0)),
        ],
        out_specs=pl.BlockSpec((N_SEG, D), lambda i: (0, 0)),
        out_shape=jax.ShapeDtypeStruct((N_SEG, D), jnp.float32),
    )(ids3d, x_tc)


def _combine_body(s_ref, t_ref, c_ref, o_ref):
    sm = s_ref[...]
    cn = c_ref[...]
    tot = sm[0] + sm[1] + t_ref[...]
    cnt = jnp.maximum((cn[0] + cn[1]).astype(jnp.float32), 1.0)
    o_ref[...] = tot / cnt[:, None]


def _combine(sums, tc_sums, counts):
    return pl.pallas_call(
        _combine_body,
        out_shape=jax.ShapeDtypeStruct((N_SEG, D), jnp.float32),
    )(sums, tc_sums, counts)


def kernel(x, batch):
    b32 = batch.astype(jnp.int32)
    pad = jnp.zeros((IDX_PAD_ROWS * GROUP - N_ROWS,), jnp.int32)
    batch2d = jnp.concatenate([b32, pad]).reshape(IDX_PAD_ROWS, GROUP)
    ids3d = b32[SC_ROWS:].reshape(TC_NBLK, 1, TC_BLK)
    tc_sums = _tc_partial(ids3d, x[SC_ROWS:])
    sums, cnts = _sc_segment_sum(x, batch2d)
    counts = cnts[:, :, ::16].reshape(NC, N_SEG)
    return _combine(sums, tc_sums, counts)


# speedup vs baseline: 1.2074x; 1.1952x over previous
import functools

import jax
import jax.numpy as jnp
from jax import lax
from jax.experimental import pallas as pl
from jax.experimental.pallas import tpu as pltpu
from jax.experimental.pallas import tpu_sc as plsc

N_ROWS = 320000
D = 128
N_SEG = 1024
NC = 2
NS = 16
NW = NC * NS
GROUP = 128
N_GROUPS = N_ROWS // GROUP
SC_GROUPS_PER_W = 64
SC_GROUPS = SC_GROUPS_PER_W * NW
SC_ROWS = SC_GROUPS * GROUP
TC_ROWS = N_ROWS - SC_ROWS
TC_BLK = 512
TC_NBLK = TC_ROWS // TC_BLK
HGROUPS_PER_W = N_GROUPS // NW
N_EXTRA = N_GROUPS - HGROUPS_PER_W * NW
IDX_PAD_ROWS = 2504
IDX_WIN = 88
SEG_PER_TILE = N_SEG // NS
HROWS = N_SEG // 8
HSEG_PER_TILE = HROWS // NS
N_QUAD = SC_GROUPS_PER_W // 4 - 1


def _sc_body(x_hbm, b_hbm, sums_hbm, cnts_hbm,
             acc, cntsq, xbuf0, xbuf1, xbuf2, xbuf3, idxs, idxh, hist,
             idbuf, fzero, gsem0, gsem1, gsem2, gsem3,
             ssem0, ssem1, ssem2, ssem3):
    c = lax.axis_index("c")
    s = lax.axis_index("s")
    wid = c * NS + s

    zeros16 = jnp.zeros((16,), jnp.float32)
    zeros16i = jnp.zeros((16,), jnp.int32)
    iota16 = lax.iota(jnp.int32, 16)

    sstart = pl.multiple_of(SC_GROUPS_PER_W * wid, 8)
    hstart = HGROUPS_PER_W * wid + jnp.minimum(wid, N_EXTRA)
    has_extra = wid < N_EXTRA
    nh = HGROUPS_PER_W + has_extra.astype(jnp.int32)
    hoff = hstart & 7
    hwstart = pl.multiple_of(hstart - hoff, 8)

    bufs = (xbuf0, xbuf1, xbuf2, xbuf3)
    gsems = (gsem0, gsem1, gsem2, gsem3)
    ssems = (ssem0, ssem1, ssem2, ssem3)

    def _gather(k, b):
        pltpu.async_copy(
            x_hbm.at[pl.ds((sstart + k) * GROUP, GROUP)], bufs[b],
            gsems[b])

    def _drain_gather(b):
        pltpu.make_async_copy(x_hbm.at[pl.ds(0, GROUP)], bufs[b],
                              gsems[b]).wait()

    def _scatter(k, b):
        pltpu.async_copy(bufs[b], acc.at[idxs.at[k]], ssems[b], add=True)

    def _drain_scatter(b):
        pltpu.make_async_copy(x_hbm.at[pl.ds(0, GROUP)], bufs[b],
                              ssems[b]).wait()

    for b in range(4):
        _gather(b, b)

    def _z_hist(k, _):
        hist[k // 8, pl.ds((k % 8) * 16, 16)] = zeros16i
        return 0
    lax.fori_loop(0, HROWS * 8, _z_hist, 0)

    def _z_fzero(k, _):
        fzero[k // 8, pl.ds((k % 8) * 16, 16)] = zeros16
        return 0
    lax.fori_loop(0, SEG_PER_TILE * 8, _z_fzero, 0)

    for j in range(8):
        idbuf[0, pl.ds(j * 16, 16)] = iota16 + (j * 16)

    seg0 = s * SEG_PER_TILE
    hseg0 = s * HSEG_PER_TILE
    pltpu.sync_copy(fzero, acc.at[pl.ds(seg0, SEG_PER_TILE)])
    pltpu.sync_copy(hist.at[pl.ds(0, HSEG_PER_TILE)],
                    cntsq.at[pl.ds(hseg0, HSEG_PER_TILE)])
    plsc.subcore_barrier()

    pltpu.sync_copy(b_hbm.at[pl.ds(sstart, SC_GROUPS_PER_W)], idxs)
    pltpu.sync_copy(b_hbm.at[pl.ds(hwstart, IDX_WIN)], idxh)

    ones16 = zeros16i + 1

    def _hist_row(r):
        for j in range(8):
            v = idxh[hoff + r, pl.ds(j * 16, 16)]
            plsc.addupdate_scatter(hist, [v >> 3, (v & 7) * 16], ones16)

    def _quad(g, _):
        k0 = 4 * g
        for b in range(4):
            _hist_row(k0 + b)
            _drain_gather(b)
            _scatter(k0 + b, b)
        for b in range(4):
            _drain_scatter(b)
            _gather(k0 + b + 4, b)
        return 0
    lax.fori_loop(0, N_QUAD, _quad, 0)

    t0 = N_QUAD * 4
    for b in range(4):
        _hist_row(t0 + b)
        _drain_gather(b)
        _scatter(t0 + b, b)
    for b in range(4):
        _drain_scatter(b)

    def _hist_rest(r, _):
        _hist_row(r)
        return 0
    lax.fori_loop(SC_GROUPS_PER_W, nh, _hist_rest, 0)

    pltpu.sync_copy(hist, cntsq.at[idbuf.at[0]], add=True)

    plsc.subcore_barrier()

    pltpu.sync_copy(acc.at[pl.ds(seg0, SEG_PER_TILE)],
                    sums_hbm.at[c, pl.ds(seg0, SEG_PER_TILE)])
    pltpu.sync_copy(cntsq.at[pl.ds(hseg0, HSEG_PER_TILE)],
                    cnts_hbm.at[c, pl.ds(hseg0, HSEG_PER_TILE)])


_sc_segment_sum = functools.partial(
    pl.kernel,
    out_type=(
        jax.ShapeDtypeStruct((NC, N_SEG, D), jnp.float32),
        jax.ShapeDtypeStruct((NC, HROWS, 128), jnp.int32),
    ),
    mesh=plsc.VectorSubcoreMesh(core_axis_name="c", subcore_axis_name="s"),
    scratch_types=[
        pltpu.VMEM_SHARED((N_SEG, D), jnp.float32),
        pltpu.VMEM_SHARED((HROWS, 128), jnp.int32),
        pltpu.VMEM((GROUP, D), jnp.float32),
        pltpu.VMEM((GROUP, D), jnp.float32),
        pltpu.VMEM((GROUP, D), jnp.float32),
        pltpu.VMEM((GROUP, D), jnp.float32),
        pltpu.VMEM((SC_GROUPS_PER_W, 128), jnp.int32),
        pltpu.VMEM((IDX_WIN, 128), jnp.int32),
        pltpu.VMEM((HROWS, 128), jnp.int32),
        pltpu.VMEM((1, 128), jnp.int32),
        pltpu.VMEM((SEG_PER_TILE, 128), jnp.float32),
        pltpu.SemaphoreType.DMA,
        pltpu.SemaphoreType.DMA,
        pltpu.SemaphoreType.DMA,
        pltpu.SemaphoreType.DMA,
        pltpu.SemaphoreType.DMA,
        pltpu.SemaphoreType.DMA,
        pltpu.SemaphoreType.DMA,
        pltpu.SemaphoreType.DMA,
    ],
    compiler_params=pltpu.CompilerParams(needs_layout_passes=False),
)(_sc_body)


def _tc_body(ids_ref, x_ref, o_ref):
    i = pl.program_id(0)

    @pl.when(i == 0)
    def _init():
        o_ref[...] = jnp.zeros_like(o_ref)

    ids = ids_ref[0, 0]
    oh = (jax.lax.broadcasted_iota(jnp.int32, (TC_BLK, N_SEG), 1)
          == ids[:, None]).astype(jnp.bfloat16)
    xb = x_ref[...].astype(jnp.bfloat16)
    o_ref[...] += lax.dot_general(
        oh, xb, (((0,), (0,)), ((), ())),
        preferred_element_type=jnp.float32)


def _tc_partial(ids3d, x):
    blk0 = SC_ROWS // TC_BLK
    return pl.pallas_call(
        _tc_body,
        grid=(TC_NBLK,),
        in_specs=[
            pl.BlockSpec((1, 1, TC_BLK), lambda i: (i, 0, 0)),
            pl.BlockSpec((TC_BLK, D), lambda i: (blk0 + i, 0)),
        ],
        out_specs=pl.BlockSpec((N_SEG, D), lambda i: (0, 0)),
        out_shape=jax.ShapeDtypeStruct((N_SEG, D), jnp.float32),
    )(ids3d, x)


def _combine_body(s_ref, t_ref, c_ref, o_ref):
    sm = s_ref[...]
    cn = c_ref[...]
    tot = sm[0] + sm[1] + t_ref[...]
    cnt = jnp.maximum((cn[0] + cn[1]).astype(jnp.float32), 1.0)
    o_ref[...] = tot / cnt[:, None]


def _combine(sums, tc_sums, counts):
    return pl.pallas_call(
        _combine_body,
        out_shape=jax.ShapeDtypeStruct((N_SEG, D), jnp.float32),
    )(sums, tc_sums, counts)


def kernel(x, batch):
    b32 = batch.astype(jnp.int32)
    pad = jnp.zeros((IDX_PAD_ROWS * GROUP - N_ROWS,), jnp.int32)
    batch2d = jnp.concatenate([b32, pad]).reshape(IDX_PAD_ROWS, GROUP)
    ids3d = b32[SC_ROWS:].reshape(TC_NBLK, 1, TC_BLK)
    tc_sums = _tc_partial(ids3d, x)
    sums, cnts = _sc_segment_sum(x, batch2d)
    counts = cnts[:, :, ::16].reshape(NC, N_SEG)
    return _combine(sums, tc_sums, counts)


# speedup vs baseline: 1.2294x; 1.0182x over previous
import functools

import jax
import jax.numpy as jnp
from jax import lax
from jax.experimental import pallas as pl
from jax.experimental.pallas import tpu as pltpu
from jax.experimental.pallas import tpu_sc as plsc

N_ROWS = 320000
D = 128
N_SEG = 1024
NC = 2
NS = 16
NW = NC * NS
GROUP = 128
N_GROUPS = N_ROWS // GROUP
SC_GROUPS_PER_W = 72
SC_GROUPS = SC_GROUPS_PER_W * NW
SC_ROWS = SC_GROUPS * GROUP
TC_ROWS = N_ROWS - SC_ROWS
TC_BLK = 512
TC_NBLK = TC_ROWS // TC_BLK
HGROUPS_PER_W = N_GROUPS // NW
N_EXTRA = N_GROUPS - HGROUPS_PER_W * NW
IDX_PAD_ROWS = 2504
IDX_WIN = 88
SEG_PER_TILE = N_SEG // NS
HROWS = N_SEG // 8
HSEG_PER_TILE = HROWS // NS
N_QUAD = SC_GROUPS_PER_W // 4 - 1


def _sc_body(x_hbm, b_hbm, sums_hbm, cnts_hbm,
             acc, cntsq, xbuf0, xbuf1, xbuf2, xbuf3, idxs, idxh, hist,
             idbuf, fzero, gsem0, gsem1, gsem2, gsem3,
             ssem0, ssem1, ssem2, ssem3):
    c = lax.axis_index("c")
    s = lax.axis_index("s")
    wid = c * NS + s

    zeros16 = jnp.zeros((16,), jnp.float32)
    zeros16i = jnp.zeros((16,), jnp.int32)
    iota16 = lax.iota(jnp.int32, 16)

    sstart = pl.multiple_of(SC_GROUPS_PER_W * wid, 8)
    hstart = HGROUPS_PER_W * wid + jnp.minimum(wid, N_EXTRA)
    has_extra = wid < N_EXTRA
    nh = HGROUPS_PER_W + has_extra.astype(jnp.int32)
    hoff = hstart & 7
    hwstart = pl.multiple_of(hstart - hoff, 8)

    bufs = (xbuf0, xbuf1, xbuf2, xbuf3)
    gsems = (gsem0, gsem1, gsem2, gsem3)
    ssems = (ssem0, ssem1, ssem2, ssem3)

    def _gather(k, b):
        pltpu.async_copy(
            x_hbm.at[pl.ds((sstart + k) * GROUP, GROUP)], bufs[b],
            gsems[b])

    def _drain_gather(b):
        pltpu.make_async_copy(x_hbm.at[pl.ds(0, GROUP)], bufs[b],
                              gsems[b]).wait()

    def _scatter(k, b):
        pltpu.async_copy(bufs[b], acc.at[idxs.at[k]], ssems[b], add=True)

    def _drain_scatter(b):
        pltpu.make_async_copy(x_hbm.at[pl.ds(0, GROUP)], bufs[b],
                              ssems[b]).wait()

    for b in range(4):
        _gather(b, b)

    def _z_hist(k, _):
        hist[k // 8, pl.ds((k % 8) * 16, 16)] = zeros16i
        return 0
    lax.fori_loop(0, HROWS * 8, _z_hist, 0)

    def _z_fzero(k, _):
        fzero[k // 8, pl.ds((k % 8) * 16, 16)] = zeros16
        return 0
    lax.fori_loop(0, SEG_PER_TILE * 8, _z_fzero, 0)

    for j in range(8):
        idbuf[0, pl.ds(j * 16, 16)] = iota16 + (j * 16)

    seg0 = s * SEG_PER_TILE
    hseg0 = s * HSEG_PER_TILE
    pltpu.sync_copy(fzero, acc.at[pl.ds(seg0, SEG_PER_TILE)])
    pltpu.sync_copy(hist.at[pl.ds(0, HSEG_PER_TILE)],
                    cntsq.at[pl.ds(hseg0, HSEG_PER_TILE)])
    plsc.subcore_barrier()

    pltpu.sync_copy(b_hbm.at[pl.ds(sstart, SC_GROUPS_PER_W)], idxs)
    pltpu.sync_copy(b_hbm.at[pl.ds(hwstart, IDX_WIN)], idxh)

    ones16 = zeros16i + 1

    def _hist_row(r):
        for j in range(8):
            v = idxh[hoff + r, pl.ds(j * 16, 16)]
            plsc.addupdate_scatter(hist, [v >> 3, (v & 7) * 16], ones16)

    def _quad(g, _):
        k0 = 4 * g
        for b in range(4):
            _hist_row(k0 + b)
            _drain_gather(b)
            _scatter(k0 + b, b)
        for b in range(4):
            _drain_scatter(b)
            _gather(k0 + b + 4, b)
        return 0
    lax.fori_loop(0, N_QUAD, _quad, 0)

    t0 = N_QUAD * 4
    for b in range(4):
        _hist_row(t0 + b)
        _drain_gather(b)
        _scatter(t0 + b, b)
    for b in range(4):
        _drain_scatter(b)

    def _hist_rest(r, _):
        _hist_row(r)
        return 0
    lax.fori_loop(SC_GROUPS_PER_W, nh, _hist_rest, 0)

    pltpu.sync_copy(hist, cntsq.at[idbuf.at[0]], add=True)

    plsc.subcore_barrier()

    pltpu.sync_copy(acc.at[pl.ds(seg0, SEG_PER_TILE)],
                    sums_hbm.at[c, pl.ds(seg0, SEG_PER_TILE)])
    pltpu.sync_copy(cntsq.at[pl.ds(hseg0, HSEG_PER_TILE)],
                    cnts_hbm.at[c, pl.ds(hseg0, HSEG_PER_TILE)])


_sc_segment_sum = functools.partial(
    pl.kernel,
    out_type=(
        jax.ShapeDtypeStruct((NC, N_SEG, D), jnp.float32),
        jax.ShapeDtypeStruct((NC, HROWS, 128), jnp.int32),
    ),
    mesh=plsc.VectorSubcoreMesh(core_axis_name="c", subcore_axis_name="s"),
    scratch_types=[
        pltpu.VMEM_SHARED((N_SEG, D), jnp.float32),
        pltpu.VMEM_SHARED((HROWS, 128), jnp.int32),
        pltpu.VMEM((GROUP, D), jnp.float32),
        pltpu.VMEM((GROUP, D), jnp.float32),
        pltpu.VMEM((GROUP, D), jnp.float32),
        pltpu.VMEM((GROUP, D), jnp.float32),
        pltpu.VMEM((SC_GROUPS_PER_W, 128), jnp.int32),
        pltpu.VMEM((IDX_WIN, 128), jnp.int32),
        pltpu.VMEM((HROWS, 128), jnp.int32),
        pltpu.VMEM((1, 128), jnp.int32),
        pltpu.VMEM((SEG_PER_TILE, 128), jnp.float32),
        pltpu.SemaphoreType.DMA,
        pltpu.SemaphoreType.DMA,
        pltpu.SemaphoreType.DMA,
        pltpu.SemaphoreType.DMA,
        pltpu.SemaphoreType.DMA,
        pltpu.SemaphoreType.DMA,
        pltpu.SemaphoreType.DMA,
        pltpu.SemaphoreType.DMA,
    ],
    compiler_params=pltpu.CompilerParams(needs_layout_passes=False),
)(_sc_body)


def _tc_body(ids_ref, x_ref, o_ref):
    i = pl.program_id(0)

    @pl.when(i == 0)
    def _init():
        o_ref[...] = jnp.zeros_like(o_ref)

    ids = ids_ref[0, 0]
    oh = (jax.lax.broadcasted_iota(jnp.int32, (TC_BLK, N_SEG), 1)
          == ids[:, None]).astype(jnp.bfloat16)
    xb = x_ref[...].astype(jnp.bfloat16)
    o_ref[...] += lax.dot_general(
        oh, xb, (((0,), (0,)), ((), ())),
        preferred_element_type=jnp.float32)


def _tc_partial(ids3d, x):
    blk0 = SC_ROWS // TC_BLK
    return pl.pallas_call(
        _tc_body,
        grid=(TC_NBLK,),
        in_specs=[
            pl.BlockSpec((1, 1, TC_BLK), lambda i: (i, 0, 0)),
            pl.BlockSpec((TC_BLK, D), lambda i: (blk0 + i, 0)),
        ],
        out_specs=pl.BlockSpec((N_SEG, D), lambda i: (0, 0)),
        out_shape=jax.ShapeDtypeStruct((N_SEG, D), jnp.float32),
    )(ids3d, x)


def _combine_body(s_ref, t_ref, c_ref, o_ref):
    sm = s_ref[...]
    cn = c_ref[...]
    tot = sm[0] + sm[1] + t_ref[...]
    cnt = jnp.maximum((cn[0] + cn[1]).astype(jnp.float32), 1.0)
    o_ref[...] = tot / cnt[:, None]


def _combine(sums, tc_sums, counts):
    return pl.pallas_call(
        _combine_body,
        out_shape=jax.ShapeDtypeStruct((N_SEG, D), jnp.float32),
    )(sums, tc_sums, counts)


def kernel(x, batch):
    b32 = batch.astype(jnp.int32)
    pad = jnp.zeros((IDX_PAD_ROWS * GROUP - N_ROWS,), jnp.int32)
    batch2d = jnp.concatenate([b32, pad]).reshape(IDX_PAD_ROWS, GROUP)
    ids3d = b32[SC_ROWS:].reshape(TC_NBLK, 1, TC_BLK)
    tc_sums = _tc_partial(ids3d, x)
    sums, cnts = _sc_segment_sum(x, batch2d)
    counts = cnts[:, :, ::16].reshape(NC, N_SEG)
    return _combine(sums, tc_sums, counts)
